# Initial kernel scaffold; baseline (speedup 1.0000x reference)
import jax
import jax.numpy as jnp
from jax.experimental import pallas as pl


def kernel(index, emb_table, W, b):
    raise NotImplementedError("write your pallas kernel here")


# speedup vs baseline: 1.3804x; 1.3804x over previous
import functools

import jax
import jax.numpy as jnp
from jax import lax
from jax.experimental import pallas as pl
from jax.experimental.pallas import tpu as pltpu
from jax.experimental.pallas import tpu_sc as plsc

NC, NS = 2, 16
NW = NC * NS
CHUNK = 128


def _gather_body(n_chunks, n_per_w, table_hbm, idx_hbm, out_hbm,
                 idx_v, rows_v, sem):
    wid = lax.axis_index("s") * NC + lax.axis_index("c")
    base = wid * n_per_w

    def chunk(j, carry):
        off = base + j * CHUNK
        pltpu.sync_copy(idx_hbm.at[pl.ds(off, CHUNK)], idx_v)
        pltpu.async_copy(table_hbm.at[idx_v], rows_v, sem).wait()
        pltpu.sync_copy(rows_v, out_hbm.at[pl.ds(off, CHUNK)])
        return carry

    lax.fori_loop(0, n_chunks, chunk, 0)


def _sc_gather(table, idx):
    n = idx.shape[0]
    d = table.shape[1]
    tile = NW * CHUNK
    n_pad = (n + tile - 1) // tile * tile
    if n_pad != n:
        idx = jnp.pad(idx, (0, n_pad - n))
    n_per_w = n_pad // NW
    n_chunks = n_per_w // CHUNK
    mesh = plsc.VectorSubcoreMesh(core_axis_name="c", subcore_axis_name="s",
                                  num_cores=NC, num_subcores=NS)
    body = functools.partial(_gather_body, n_chunks, n_per_w)
    out = pl.kernel(
        body,
        out_type=jax.ShapeDtypeStruct((n_pad, d), jnp.float32),
        mesh=mesh,
        scratch_types=[
            pltpu.VMEM((CHUNK,), jnp.int32),
            pltpu.VMEM((CHUNK, d), jnp.float32),
            pltpu.SemaphoreType.DMA,
        ],
    )(table, idx)
    return out[:n] if n_pad != n else out


def _mm_body(x_ref, w_ref, b_ref, o_ref):
    x = x_ref[...]
    y = jnp.dot(x, w_ref[...], preferred_element_type=jnp.float32) + b_ref[...]
    c = 0.7978845608028654
    o_ref[...] = 0.5 * y * (1.0 + jnp.tanh(c * (y + 0.044715 * (y * y * y))))


def _tc_transform(x, wt, b2):
    n, d_in = x.shape
    d_out = wt.shape[1]
    blk = 2048
    assert n % blk == 0
    return pl.pallas_call(
        _mm_body,
        grid=(n // blk,),
        in_specs=[
            pl.BlockSpec((blk, d_in), lambda i: (i, 0)),
            pl.BlockSpec((d_in, d_out), lambda i: (0, 0)),
            pl.BlockSpec((1, d_out), lambda i: (0, 0)),
        ],
        out_specs=pl.BlockSpec((blk, d_out), lambda i: (i, 0)),
        out_shape=jax.ShapeDtypeStruct((n, d_out), jnp.float32),
    )(x, wt, b2)


def kernel(index, emb_table, W, b):
    bsz, seq = index.shape
    n = bsz * seq
    idx = index.reshape(n).astype(jnp.int32)
    x = _sc_gather(emb_table, idx)
    y = _tc_transform(x, W.T, b.reshape(1, -1))
    return y.reshape(bsz, seq, W.shape[0])


# speedup vs baseline: 1.5168x; 1.0988x over previous
import functools

import jax
import jax.numpy as jnp
from jax import lax
from jax.experimental import pallas as pl
from jax.experimental.pallas import tpu as pltpu
from jax.experimental.pallas import tpu_sc as plsc

NC, NS = 2, 16
NW = NC * NS
CHUNK = 128


def _gather_body(n_chunks, table_hbm, idx_hbm, out_hbm,
                 idx_v, rows0, rows1, sg0, sg1, ss0, ss1):
    wid = lax.axis_index("s") * NC + lax.axis_index("c")
    base = wid * n_chunks * CHUNK
    pltpu.sync_copy(idx_hbm.at[wid], idx_v)

    rows = (rows0, rows1)
    sg = (sg0, sg1)
    ss = (ss0, ss1)

    def start_gather(j, b):
        pltpu.async_copy(table_hbm.at[idx_v.at[j]], rows[b], sg[b])

    def wait_gather(j, b):
        pltpu.make_async_copy(table_hbm.at[idx_v.at[j]], rows[b], sg[b]).wait()

    def start_store(j, b):
        pltpu.async_copy(rows[b], out_hbm.at[pl.ds(base + j * CHUNK, CHUNK)], ss[b])

    def wait_store(j, b):
        pltpu.make_async_copy(
            rows[b], out_hbm.at[pl.ds(base + j * CHUNK, CHUNK)], ss[b]).wait()

    start_gather(0, 0)
    start_gather(1, 1)

    def pair(g, carry):
        for b in range(2):
            j = 2 * g + b
            wait_gather(j, b)
            start_store(j, b)
            wait_store(j, b)
            start_gather(j + 2, b)
        return carry

    lax.fori_loop(0, n_chunks // 2 - 1, pair, 0)
    for b in range(2):
        j = n_chunks - 2 + b
        wait_gather(j, b)
        start_store(j, b)
        wait_store(j, b)


def _sc_gather(table, idx3d):
    _, n_chunks, _ = idx3d.shape
    d = table.shape[1]
    n = NW * n_chunks * CHUNK
    mesh = plsc.VectorSubcoreMesh(core_axis_name="c", subcore_axis_name="s",
                                  num_cores=NC, num_subcores=NS)
    body = functools.partial(_gather_body, n_chunks)
    return pl.kernel(
        body,
        out_type=jax.ShapeDtypeStruct((n, d), jnp.float32),
        mesh=mesh,
        scratch_types=[
            pltpu.VMEM((n_chunks, CHUNK), jnp.int32),
            pltpu.VMEM((CHUNK, d), jnp.float32),
            pltpu.VMEM((CHUNK, d), jnp.float32),
            pltpu.SemaphoreType.DMA,
            pltpu.SemaphoreType.DMA,
            pltpu.SemaphoreType.DMA,
            pltpu.SemaphoreType.DMA,
        ],
    )(table, idx3d)


def _mm_body(x_ref, w_ref, b_ref, o_ref):
    x = x_ref[...]
    y = jnp.dot(x, w_ref[...], preferred_element_type=jnp.float32) + b_ref[...]
    c = 0.7978845608028654
    o_ref[...] = 0.5 * y * (1.0 + jnp.tanh(c * (y + 0.044715 * (y * y * y))))


def _tc_transform(x, wt, b2):
    n, d_in = x.shape
    d_out = wt.shape[1]
    blk = 2048
    assert n % blk == 0
    return pl.pallas_call(
        _mm_body,
        grid=(n // blk,),
        in_specs=[
            pl.BlockSpec((blk, d_in), lambda i: (i, 0)),
            pl.BlockSpec((d_in, d_out), lambda i: (0, 0)),
            pl.BlockSpec((1, d_out), lambda i: (0, 0)),
        ],
        out_specs=pl.BlockSpec((blk, d_out), lambda i: (i, 0)),
        out_shape=jax.ShapeDtypeStruct((n, d_out), jnp.float32),
    )(x, wt, b2)


def kernel(index, emb_table, W, b):
    bsz, seq = index.shape
    n = bsz * seq
    idx = index.reshape(n).astype(jnp.int32)
    tile = NW * CHUNK * 2
    n_pad = (n + tile - 1) // tile * tile
    if n_pad != n:
        idx = jnp.pad(idx, (0, n_pad - n))
    x = _sc_gather(emb_table, idx.reshape(NW, n_pad // (NW * CHUNK), CHUNK))
    if n_pad != n:
        x = x[:n]
    y = _tc_transform(x, W.T, b.reshape(1, -1))
    return y.reshape(bsz, seq, W.shape[0])
